# Initial kernel scaffold; baseline (speedup 1.0000x reference)
import jax
import jax.numpy as jnp
from jax.experimental import pallas as pl


def kernel(x, colors, ctx_params, body_params, lstm_params, fc1_w, fc1_b, lift_w, lift_b):
    raise NotImplementedError("write your pallas kernel here")


# speedup vs baseline: 1.4026x; 1.4026x over previous
import functools

import jax
import jax.numpy as jnp
import numpy as np
from jax import lax
from jax.experimental import pallas as pl
from jax.experimental.pallas import tpu as pltpu

_NCPAD = 152
_NG = _NCPAD // 8


def _hit_kernel(cc_ref, sem_ref, out_ref, code_ref):
    j = pl.program_id(1)

    @pl.when(j == 0)
    def _():
        out_ref[...] = jnp.zeros_like(out_ref)

    s = sem_ref[0]
    code_ref[...] = s[0] * 65536.0 + s[1] * 256.0 + s[2]

    def chunk(r, _):
        tile = code_ref[pl.ds(r * 8, 8), :]
        px = [jnp.broadcast_to(tile[rr:rr + 1, :], (8, 256)) for rr in range(8)]
        for g in range(_NG):
            cc_g = cc_ref[g * 8:(g + 1) * 8, :]
            acc = out_ref[0, g * 8:(g + 1) * 8, :]
            for rr in range(8):
                acc = jnp.where(px[rr] == cc_g, 1.0, acc)
            out_ref[0, g * 8:(g + 1) * 8, :] = acc
        return 0

    lax.fori_loop(0, 32, chunk, 0)


def _hit_pallas(sem, colors):
    B = sem.shape[0]
    half = B // 2
    c = colors.astype(jnp.float32)
    ccode = c[:, 0] * 65536.0 + c[:, 1] * 256.0 + c[:, 2]
    ccode = jnp.concatenate([ccode, -jnp.ones((_NCPAD - 150,), jnp.float32)])
    cc_bc = jnp.broadcast_to(ccode[:, None], (_NCPAD, 256))

    out = pl.pallas_call(
        _hit_kernel,
        grid=(2, half),
        in_specs=[
            pl.BlockSpec((_NCPAD, 256), lambda c_, j: (0, 0)),
            pl.BlockSpec((1, 3, 256, 256), lambda c_, j, h=half: (c_ * h + j, 0, 0, 0)),
        ],
        out_specs=pl.BlockSpec((1, _NCPAD, 256), lambda c_, j: (c_, 0, 0)),
        out_shape=jax.ShapeDtypeStruct((2, _NCPAD, 256), jnp.float32),
        scratch_shapes=[pltpu.VMEM((256, 256), jnp.float32)],
        compiler_params=pltpu.CompilerParams(
            dimension_semantics=("parallel", "arbitrary")),
    )(cc_bc, sem)
    return jnp.max(out, axis=(0, 2))[:150]


def _conv(x, w, b, stride, pad):
    y = lax.conv_general_dilated(x, w, (stride, stride), [(pad, pad), (pad, pad)],
                                 dimension_numbers=('NCHW', 'OIHW', 'NCHW'))
    return y + b[None, :, None, None]


def _maxpool(x):
    return lax.reduce_window(x, -jnp.inf, lax.max, (1, 1, 3, 3), (1, 1, 2, 2), 'VALID')


def _alexnet(p, x):
    x = jax.nn.relu(_conv(x, p['c1w'], p['c1b'], 4, 2)); x = _maxpool(x)
    x = jax.nn.relu(_conv(x, p['c2w'], p['c2b'], 1, 2)); x = _maxpool(x)
    x = jax.nn.relu(_conv(x, p['c3w'], p['c3b'], 1, 1))
    x = jax.nn.relu(_conv(x, p['c4w'], p['c4b'], 1, 1))
    x = jax.nn.relu(_conv(x, p['c5w'], p['c5b'], 1, 1)); x = _maxpool(x)
    x = lax.reduce_window(x, 0.0, lax.add, (1, 1, 2, 2), (1, 1, 1, 1), 'VALID') * 0.25
    x = x.reshape(x.shape[0], -1)
    x = jax.nn.relu(x @ p['f1w'].T + p['f1b'])
    x = jax.nn.relu(x @ p['f2w'].T + p['f2b'])
    return x @ p['f3w'].T + p['f3b']


def _lstm_dir(xs, wih, whh, bih, bhh, reverse):
    H = whh.shape[1]
    pre = jnp.einsum('tnd,gd->tng', xs, wih) + bih + bhh

    def step(carry, u):
        h, c = carry
        g = u + h @ whh.T
        i, f, gg, o = jnp.split(g, 4, axis=-1)
        c = jax.nn.sigmoid(f) * c + jax.nn.sigmoid(i) * jnp.tanh(gg)
        h = jax.nn.sigmoid(o) * jnp.tanh(c)
        return (h, c), h

    init = (jnp.zeros((xs.shape[1], H), xs.dtype), jnp.zeros((xs.shape[1], H), xs.dtype))
    _, hs = lax.scan(step, init, pre, reverse=reverse)
    return hs


def _bilstm_layer(xs, p, l):
    hf = _lstm_dir(xs, p['wih%d' % l][0], p['whh%d' % l][0], p['bih%d' % l][0], p['bhh%d' % l][0], False)
    hb = _lstm_dir(xs, p['wih%d' % l][1], p['whh%d' % l][1], p['bih%d' % l][1], p['bhh%d' % l][1], True)
    return jnp.concatenate([hf, hb], axis=-1)


def kernel(x, colors, ctx_params, body_params, lstm_params, fc1_w, fc1_b, lift_w, lift_b):
    B = x.shape[0]
    context = x[:, :, :256, :]
    body = x[:, :, 256:512, :]
    sem = x[:, :, 512:768, :]

    hit = _hit_pallas(sem, colors)

    cf = _alexnet(ctx_params, context)
    bf = _alexnet(body_params, body)

    encoder = jnp.broadcast_to(hit[None, None, :], (B, 1, 150))
    seq = jnp.transpose(encoder, (0, 2, 1))
    h = _bilstm_layer(seq, lstm_params, 0)
    h = _bilstm_layer(h, lstm_params, 1)
    fs = jnp.mean(jnp.transpose(h, (0, 2, 1)), axis=-2)
    feat_sem = fs @ lift_w.T + lift_b
    fuse = jnp.concatenate([cf, bf], axis=1) @ fc1_w.T + fc1_b
    return fuse, feat_sem
